# baseline (speedup 1.0000x reference)
import functools

import jax
import jax.numpy as jnp
from jax import lax
from jax.experimental import pallas as pl
from jax.experimental.pallas import tpu as pltpu
from jax.experimental.pallas import tpu_sc as plsc

_NC = 2
_NS = 16
_NW = _NC * _NS
_D = 64
_T = 50
_BB = 128


def _make_gather():
    mesh = plsc.VectorSubcoreMesh(
        core_axis_name="c",
        subcore_axis_name="s",
        num_cores=_NC,
        num_subcores=_NS,
    )

    @functools.partial(
        pl.kernel,
        out_type=jax.ShapeDtypeStruct((_T, 8, _NW, 8, 128), jnp.float32),
        mesh=mesh,
        scratch_types=[
            pltpu.VMEM((_T, _BB), jnp.int32),
            pltpu.VMEM((_T, _BB), jnp.int32),
            pltpu.VMEM((_BB, 128), jnp.float32),
            pltpu.VMEM((8, 8, 128), jnp.float32),
            pltpu.SemaphoreType.DMA,
            pltpu.SemaphoreType.DMA,
        ],
        compiler_params=pltpu.CompilerParams(needs_layout_passes=False),
    )
    def gather(idx_hbm, off_hbm, table_hbm, out_hbm,
               idx_v, off_v, staged, slab, gsem, osem):
        wid = lax.axis_index("s") * _NC + lax.axis_index("c")
        pltpu.sync_copy(idx_hbm.at[wid], idx_v)
        pltpu.sync_copy(off_hbm.at[wid], off_v)
        lane = lax.iota(jnp.int32, 16)

        @pl.loop(0, _T)
        def _t_loop(t):
            copies = []
            for bg in range(8):
                idx_vec = idx_v[t, pl.ds(bg * 16, 16)]
                copies.append(pltpu.async_copy(
                    table_hbm.at[idx_vec],
                    staged.at[pl.ds(bg * 16, 16)],
                    gsem))
            for cp in copies:
                cp.wait()
            for bg in range(8):
                row_vec = lane + (bg * 16)
                o_vec = off_v[t, pl.ds(bg * 16, 16)]
                for d in range(_D):
                    val = plsc.load_gather(staged, [row_vec, o_vec + d])
                    slab[d // 8, d % 8, pl.ds(bg * 16, 16)] = val
            pltpu.async_copy(slab, out_hbm.at[t, :, wid], osem).wait()

    return gather


_gather = _make_gather()


def kernel(x, weight):
    table = weight.reshape(500000, 128)
    xw = jnp.transpose(x.reshape(_NW, _BB, _T), (0, 2, 1))
    idx2 = xw >> 1
    off = (xw & 1) * 64
    p = _gather(idx2, off, table)
    return p.transpose(2, 4, 0, 1, 3).reshape(4096, _T, _D)


# speedup vs baseline: 1.0481x; 1.0481x over previous
import functools

import jax
import jax.numpy as jnp
from jax import lax
from jax.experimental import pallas as pl
from jax.experimental.pallas import tpu as pltpu
from jax.experimental.pallas import tpu_sc as plsc

_NC = 2
_NS = 16
_NW = _NC * _NS
_NE = 1000000
_D = 64
_T = 50
_BB = 128


def _make_gather():
    mesh = plsc.VectorSubcoreMesh(
        core_axis_name="c",
        subcore_axis_name="s",
        num_cores=_NC,
        num_subcores=_NS,
    )

    @functools.partial(
        pl.kernel,
        out_type=jax.ShapeDtypeStruct((_T, 8, _NW, 8, 128), jnp.float32),
        mesh=mesh,
        scratch_types=[
            pltpu.VMEM((_T, _BB), jnp.int32),
            pltpu.VMEM((_BB, _D), jnp.float32),
            pltpu.VMEM((_BB, _D), jnp.float32),
            pltpu.VMEM((8, 8, 128), jnp.float32),
            pltpu.VMEM((8, 8, 128), jnp.float32),
            pltpu.SemaphoreType.DMA,
            pltpu.SemaphoreType.DMA,
            pltpu.SemaphoreType.DMA,
            pltpu.SemaphoreType.DMA,
        ],
        compiler_params=pltpu.CompilerParams(
            use_tc_tiling_on_sc=False, needs_layout_passes=False),
    )
    def gather(idx_hbm, table_hbm, out_hbm,
               idx_v, st0, st1, sl0, sl1, g0, g1, o0, o1):
        wid = lax.axis_index("s") * _NC + lax.axis_index("c")
        pltpu.sync_copy(idx_hbm.at[wid], idx_v)
        lane = lax.iota(jnp.int32, 16)
        zero = lane * 0
        st = (st0, st1)
        sl = (sl0, sl1)
        gs = (g0, g1)
        os = (o0, o1)

        def fire_gathers(t, p):
            for bg in range(8):
                idx_vec = idx_v[t, pl.ds(bg * 16, 16)]
                pltpu.async_copy(
                    table_hbm.at[idx_vec],
                    st[p].at[pl.ds(bg * 16, 16)],
                    gs[p])

        def wait_gathers(p):
            pltpu.make_async_copy(
                table_hbm.at[pl.ds(0, _BB)], st[p], gs[p]).wait()

        def wait_out(p):
            pltpu.make_async_copy(sl[p], out_hbm.at[0, :, wid], os[p]).wait()

        def assemble(p):
            for bg in range(8):
                row_vec = lane + (bg * 16)
                for d in range(_D):
                    val = plsc.load_gather(st[p], [row_vec, zero + d])
                    sl[p][d // 8, d % 8, pl.ds(bg * 16, 16)] = val

        fire_gathers(0, 0)

        @pl.loop(0, _T // 2)
        def _t_loop(i):
            t = i * 2
            for p in range(2):
                wait_gathers(p)

                nxt_exists = (t + p + 1) < _T

                @pl.when(nxt_exists)
                def _():
                    fire_gathers(
                        jnp.minimum(t + p + 1, _T - 1), 1 - p)

                @pl.when(i > 0)
                def _():
                    wait_out(p)

                assemble(p)
                pltpu.async_copy(sl[p], out_hbm.at[t + p, :, wid], os[p])

        wait_out(0)
        wait_out(1)

    return gather


_gather = _make_gather()


def kernel(x, weight):
    xw = jnp.transpose(x.reshape(_NW, _BB, _T), (0, 2, 1))
    p = _gather(xw, weight)
    return p.transpose(2, 4, 0, 1, 3).reshape(4096, _T, _D)


# speedup vs baseline: 1.0496x; 1.0015x over previous
import functools

import jax
import jax.numpy as jnp
from jax import lax
from jax.experimental import pallas as pl
from jax.experimental.pallas import tpu as pltpu
from jax.experimental.pallas import tpu_sc as plsc

_NC = 2
_NS = 16
_NW = _NC * _NS
_NE = 1000000
_D = 64
_T = 50
_BB = 128


def _make_gather():
    mesh = plsc.VectorSubcoreMesh(
        core_axis_name="c",
        subcore_axis_name="s",
        num_cores=_NC,
        num_subcores=_NS,
    )

    @functools.partial(
        pl.kernel,
        out_type=jax.ShapeDtypeStruct((_T, 8, _NW, 8, 128), jnp.float32),
        mesh=mesh,
        scratch_types=[
            pltpu.VMEM((_BB,), jnp.int32),
            pltpu.VMEM((_BB,), jnp.int32),
            pltpu.VMEM((_BB, _D), jnp.float32),
            pltpu.VMEM((_BB, _D), jnp.float32),
            pltpu.VMEM((8, 8, 128), jnp.float32),
            pltpu.VMEM((8, 8, 128), jnp.float32),
            pltpu.SemaphoreType.DMA,
            pltpu.SemaphoreType.DMA,
            pltpu.SemaphoreType.DMA,
            pltpu.SemaphoreType.DMA,
            pltpu.SemaphoreType.DMA,
            pltpu.SemaphoreType.DMA,
        ],
        compiler_params=pltpu.CompilerParams(
            use_tc_tiling_on_sc=False, needs_layout_passes=False),
    )
    def gather(idx_hbm, table_hbm, out_hbm,
               ib0, ib1, st0, st1, sl0, sl1,
               is0, is1, gs0, gs1, os0, os1):
        wid = lax.axis_index("s") * _NC + lax.axis_index("c")
        lane = lax.iota(jnp.int32, 16)
        zero = lane * 0
        ib = (ib0, ib1)
        st = (st0, st1)
        sl = (sl0, sl1)
        isem = (is0, is1)
        gsem = (gs0, gs1)
        osem = (os0, os1)

        def wait_idx(p):
            pltpu.make_async_copy(idx_hbm.at[wid, 0], ib[p], isem[p]).wait()

        def wait_gather(p):
            pltpu.make_async_copy(
                table_hbm.at[pl.ds(0, _BB)], st[p], gsem[p]).wait()

        def wait_out(p):
            pltpu.make_async_copy(sl[p], out_hbm.at[0, :, wid], osem[p]).wait()

        def assemble(p):
            for bg in range(8):
                row_vec = lane + (bg * 16)
                for d in range(_D):
                    val = plsc.load_gather(st[p], [row_vec, zero + d])
                    sl[p][d // 8, d % 8, pl.ds(bg * 16, 16)] = val

        pltpu.sync_copy(idx_hbm.at[wid, 0], ib0)
        pltpu.async_copy(table_hbm.at[ib0], st0, gs0)
        pltpu.async_copy(idx_hbm.at[wid, 1], ib1, is1)

        @pl.loop(0, _T // 2)
        def _t_loop(j):
            for p in range(2):
                t = j * 2 + p
                nxt1 = jnp.minimum(t + 1, _T - 1)
                nxt2 = jnp.minimum(t + 2, _T - 1)
                q = 1 - p
                wait_idx(q)
                wait_gather(p)
                pltpu.async_copy(table_hbm.at[ib[q]], st[q], gsem[q])
                pltpu.async_copy(idx_hbm.at[wid, nxt2], ib[p], isem[p])

                @pl.when(j > 0)
                def _():
                    wait_out(p)

                assemble(p)
                pltpu.async_copy(sl[p], out_hbm.at[t, :, wid], osem[p])

        wait_gather(0)
        wait_idx(1)
        wait_out(0)
        wait_out(1)

    return gather


_gather = _make_gather()


def kernel(x, weight):
    xw = jnp.transpose(x.reshape(_NW, _BB, _T), (0, 2, 1))
    p = _gather(xw, weight)
    return p.transpose(2, 4, 0, 1, 3).reshape(4096, _T, _D)


# speedup vs baseline: 1.5666x; 1.4925x over previous
import functools

import jax
import jax.numpy as jnp
from jax import lax
from jax.experimental import pallas as pl
from jax.experimental.pallas import tpu as pltpu
from jax.experimental.pallas import tpu_sc as plsc

_NC = 2
_NS = 16
_NW = _NC * _NS

_NE = 1000000
_B = 4096 * 50
_D = 64
_BPW = _B // _NW
_CHUNK = 800
_NCHUNK = _BPW // _CHUNK

_HALF = 524288
_TBLK = 1024


def _tc_transpose():
    def body(a_ref, b_ref, out_ref):
        out_ref[...] = jnp.concatenate(
            [jnp.transpose(a_ref[...]), jnp.transpose(b_ref[...])], axis=1)

    nblk = _HALF // _TBLK
    last = (_NE - 1) // _TBLK
    return pl.pallas_call(
        body,
        grid=(nblk,),
        in_specs=[
            pl.BlockSpec((_D, _TBLK), lambda j: (0, j)),
            pl.BlockSpec(
                (_D, _TBLK),
                lambda j, n=nblk, m=last: (0, jnp.minimum(j + n, m))),
        ],
        out_specs=pl.BlockSpec((_TBLK, 128), lambda j: (j, 0)),
        out_shape=jax.ShapeDtypeStruct((_HALF, 128), jnp.float32),
    )


_transpose = _tc_transpose()


def _make_gather():
    mesh = plsc.VectorSubcoreMesh(
        core_axis_name="c",
        subcore_axis_name="s",
        num_cores=_NC,
        num_subcores=_NS,
    )

    @functools.partial(
        pl.kernel,
        out_type=jax.ShapeDtypeStruct((_B, _D), jnp.float32),
        mesh=mesh,
        scratch_types=[
            pltpu.VMEM((_CHUNK,), jnp.int32),
            pltpu.VMEM((_CHUNK,), jnp.int32),
            pltpu.VMEM((_CHUNK, _D), jnp.float32),
            pltpu.VMEM((_CHUNK, _D), jnp.float32),
            pltpu.SemaphoreType.DMA,
            pltpu.SemaphoreType.DMA,
            pltpu.SemaphoreType.DMA,
            pltpu.SemaphoreType.DMA,
        ],
        compiler_params=pltpu.CompilerParams(use_tc_tiling_on_sc=False),
    )
    def gather(idx_hbm, table_hbm, out_hbm,
               idx0, idx1, rows0, rows1, gsem0, gsem1, osem0, osem1):
        wid = lax.axis_index("s") * _NC + lax.axis_index("c")
        base = wid * _BPW
        idx = (idx0, idx1)
        rows = (rows0, rows1)
        gsem = (gsem0, gsem1)
        osem = (osem0, osem1)

        gat = [None, None]
        outcp = [None, None]
        pltpu.sync_copy(idx_hbm.at[wid, 0], idx0)
        gat[0] = pltpu.async_copy(table_hbm.at[idx0], rows0, gsem0)
        for g in range(_NCHUNK):
            b = g & 1
            nb = 1 - b
            if g + 1 < _NCHUNK:
                pltpu.sync_copy(idx_hbm.at[wid, g + 1], idx[nb])
                if outcp[nb] is not None:
                    outcp[nb].wait()
                gat[nb] = pltpu.async_copy(
                    table_hbm.at[idx[nb]], rows[nb], gsem[nb])
            gat[b].wait()
            outcp[b] = pltpu.async_copy(
                rows[b], out_hbm.at[pl.ds(base + g * _CHUNK, _CHUNK)],
                osem[b])
        outcp[0].wait()
        outcp[1].wait()

    return gather


_gather = _make_gather()


def kernel(x, weight):
    wt = weight.T
    table = _transpose(wt, wt).reshape(2 * _HALF, _D)
    xi = jnp.where(x < _HALF, x * 2, (x - _HALF) * 2 + 1)
    idx = xi.reshape(_NW, _NCHUNK, _CHUNK)
    out = _gather(idx, table)
    return out.reshape(x.shape[0], x.shape[1], _D)
